# baseline (speedup 1.0000x reference)
import jax
import jax.numpy as jnp
from jax.experimental import pallas as pl
from jax.experimental.pallas import tpu as pltpu

TOKEN_BLOCK = 512


def _router_kernel(x_ref, w_ref, noise_ref, out_ref):
    logits = jax.lax.dot_general(
        x_ref[...],
        w_ref[...],
        dimension_numbers=(((1,), (1,)), ((), ())),
        preferred_element_type=jnp.float32,
    )
    out_ref[...] = logits + noise_ref[...]


def kernel(x, W, noise):
    tokens, d_model = x.shape
    n_experts = W.shape[0]
    grid = (tokens // TOKEN_BLOCK,)
    return pl.pallas_call(
        _router_kernel,
        grid=grid,
        in_specs=[
            pl.BlockSpec((TOKEN_BLOCK, d_model), lambda i: (i, 0)),
            pl.BlockSpec((n_experts, d_model), lambda i: (0, 0)),
            pl.BlockSpec((TOKEN_BLOCK, n_experts), lambda i: (i, 0)),
        ],
        out_specs=pl.BlockSpec((TOKEN_BLOCK, n_experts), lambda i: (i, 0)),
        out_shape=jax.ShapeDtypeStruct((tokens, n_experts), jnp.float32),
        compiler_params=pltpu.CompilerParams(
            dimension_semantics=("parallel",),
        ),
    )(x, W, noise)


# speedup vs baseline: 1.1432x; 1.1432x over previous
import jax
import jax.numpy as jnp
from jax.experimental import pallas as pl
from jax.experimental.pallas import tpu as pltpu

TOKEN_BLOCK = 1024
N_SPLITS = 4


def _router_kernel(x0_ref, x1_ref, x2_ref, x3_ref, w_ref, noise_ref, out_ref):
    d_split = x0_ref.shape[1]
    acc = noise_ref[...]
    for j, x_ref in enumerate((x0_ref, x1_ref, x2_ref, x3_ref)):
        w_j = w_ref[:, j * d_split:(j + 1) * d_split]
        acc = acc + jax.lax.dot_general(
            x_ref[...],
            w_j,
            dimension_numbers=(((1,), (1,)), ((), ())),
            preferred_element_type=jnp.float32,
        )
    out_ref[...] = acc


def kernel(x, W, noise):
    tokens, d_model = x.shape
    n_experts = W.shape[0]
    d_split = d_model // N_SPLITS
    grid = (tokens // TOKEN_BLOCK,)

    def x_spec(j):
        return pl.BlockSpec((TOKEN_BLOCK, d_split), lambda i, j=j: (i, j))

    return pl.pallas_call(
        _router_kernel,
        grid=grid,
        in_specs=[x_spec(j) for j in range(N_SPLITS)] + [
            pl.BlockSpec((n_experts, d_model), lambda i: (0, 0)),
            pl.BlockSpec((TOKEN_BLOCK, n_experts), lambda i: (i, 0)),
        ],
        out_specs=pl.BlockSpec((TOKEN_BLOCK, n_experts), lambda i: (i, 0)),
        out_shape=jax.ShapeDtypeStruct((tokens, n_experts), jnp.float32),
        compiler_params=pltpu.CompilerParams(
            dimension_semantics=("arbitrary",),
        ),
    )(x, x, x, x, W, noise)
